# baseline (speedup 1.0000x reference)
import jax
import jax.numpy as jnp
from jax import lax
from jax.experimental import pallas as pl
from jax.experimental.pallas import tpu as pltpu
from jax.experimental.pallas import tpu_sc as plsc

_NC = 2
_NS = 16
_NW = _NC * _NS
_EB = 80
_CH = 16
_APAD = 48


def _deg_body(dst2d, out, didx, ones_v, zbuf, accum):
    c = lax.axis_index("c")
    s = lax.axis_index("s")
    wid = c * _NS + s
    npad = accum.shape[0]
    rpt = npad // _NS
    rows = dst2d.shape[0] // _NW

    def fill_ones(i, _):
        ones_v[pl.ds(i * 16, 16)] = jnp.ones((16,), jnp.float32)
        return 0

    lax.fori_loop(0, _EB // 16, fill_ones, 0)

    def fill_zero(i, _):
        zbuf[pl.ds(i * 16, 16)] = jnp.zeros((16,), jnp.float32)
        return 0

    lax.fori_loop(0, rpt // 16, fill_zero, 0)

    pltpu.sync_copy(zbuf, accum.at[pl.ds(s * rpt, rpt)])
    pltpu.sync_copy(dst2d.at[pl.ds(wid * rows, rows)], didx)
    plsc.subcore_barrier()

    def body(j, _):
        pltpu.sync_copy(ones_v, accum.at[didx.at[j]], add=True)
        return 0

    lax.fori_loop(0, rows, body, 0)
    plsc.subcore_barrier()
    pltpu.sync_copy(accum.at[pl.ds(s * rpt, rpt)], out.at[c, pl.ds(s * rpt, rpt)])


def _make_deg(nrows, npad):
    mesh = plsc.VectorSubcoreMesh(core_axis_name="c", subcore_axis_name="s")
    return pl.kernel(
        _deg_body,
        out_type=jax.ShapeDtypeStruct((_NC, npad), jnp.float32),
        mesh=mesh,
        scratch_types=[
            pltpu.VMEM((nrows // _NW, _EB), jnp.int32),
            pltpu.VMEM((_EB,), jnp.float32),
            pltpu.VMEM((npad // _NS,), jnp.float32),
            pltpu.VMEM_SHARED((npad,), jnp.float32),
        ],
    )


def _agg_body(hs, src2d, dst2d, out, sc0, sc1, dc0, dc1,
              b0, b1, b2, b3, accum, g0, g1, g2, g3, t0, t1, t2, t3):
    c = lax.axis_index("c")
    s = lax.axis_index("s")
    wid = c * _NS + s
    na = accum.shape[0]
    rpt = na // _NS
    win = rpt + (8 - rpt % 8) % 8
    start8 = (s * rpt // 8) * 8
    rows = src2d.shape[0] // _NW
    nch = rows // _CH
    base = wid * rows
    scb = (sc0, sc1)
    dcb = (dc0, dc1)
    bufs = (b0, b1, b2, b3)
    gsem = (g0, g1, g2, g3)
    tsem = (t0, t1, t2, t3)

    def zfill(i, _):
        for cb in range(8):
            b0[i, pl.ds(cb * 16, 16)] = jnp.zeros((16,), jnp.float32)
        return 0

    lax.fori_loop(0, _EB, zfill, 0)
    zfull, zrem = win // _EB, win % _EB
    for i in range(zfull):
        pltpu.sync_copy(b0, accum.at[pl.ds(start8 + i * _EB, _EB)])
    if zrem:
        pltpu.sync_copy(
            b0.at[pl.ds(0, zrem)], accum.at[pl.ds(start8 + zfull * _EB, zrem)]
        )
    plsc.subcore_barrier()

    pltpu.sync_copy(src2d.at[pl.ds(base, _CH)], sc0)
    pltpu.sync_copy(dst2d.at[pl.ds(base, _CH)], dc0)
    pltpu.async_copy(hs.at[sc0.at[0]], b0, g0)
    pltpu.async_copy(hs.at[sc0.at[1]], b1, g1)

    for k in range(nch):
        cur_s, cur_d = scb[k % 2], dcb[k % 2]
        nxt_s, nxt_d = scb[(k + 1) % 2], dcb[(k + 1) % 2]
        if k + 1 < nch:
            pltpu.sync_copy(src2d.at[pl.ds(base + (k + 1) * _CH, _CH)], nxt_s)
            pltpu.sync_copy(dst2d.at[pl.ds(base + (k + 1) * _CH, _CH)], nxt_d)
        for jj in range(_CH):
            j = k * _CH + jj
            b = j % 4
            pltpu.make_async_copy(hs.at[cur_s.at[jj]], bufs[b], gsem[b]).wait()
            pltpu.async_copy(bufs[b], accum.at[cur_d.at[jj]], tsem[b], add=True)
            if j >= 2:
                pb = (j - 2) % 4
                pltpu.make_async_copy(
                    bufs[pb], accum.at[cur_d.at[jj]], tsem[pb]
                ).wait()
            if j + 2 < rows:
                nj = jj + 2
                tb = (j + 2) % 4
                idxrow = cur_s.at[nj] if nj < _CH else nxt_s.at[nj - _CH]
                pltpu.async_copy(hs.at[idxrow], bufs[tb], gsem[tb])

    last_d = dcb[(nch - 1) % 2]
    for j in (rows - 2, rows - 1):
        b = j % 4
        pltpu.make_async_copy(bufs[b], accum.at[last_d.at[_CH - 1]], tsem[b]).wait()

    plsc.subcore_barrier()
    pltpu.sync_copy(
        accum.at[pl.ds(start8, win)], out.at[c, pl.ds(start8, win)]
    )


def _make_agg(na, d, nrows):
    mesh = plsc.VectorSubcoreMesh(core_axis_name="c", subcore_axis_name="s")
    return pl.kernel(
        _agg_body,
        out_type=jax.ShapeDtypeStruct((_NC, na, d), jnp.float32),
        mesh=mesh,
        scratch_types=[
            pltpu.VMEM((_CH, _EB), jnp.int32),
            pltpu.VMEM((_CH, _EB), jnp.int32),
            pltpu.VMEM((_CH, _EB), jnp.int32),
            pltpu.VMEM((_CH, _EB), jnp.int32),
            pltpu.VMEM((_EB, d), jnp.float32),
            pltpu.VMEM((_EB, d), jnp.float32),
            pltpu.VMEM((_EB, d), jnp.float32),
            pltpu.VMEM((_EB, d), jnp.float32),
            pltpu.VMEM_SHARED((na, d), jnp.float32),
            pltpu.SemaphoreType.DMA,
            pltpu.SemaphoreType.DMA,
            pltpu.SemaphoreType.DMA,
            pltpu.SemaphoreType.DMA,
            pltpu.SemaphoreType.DMA,
            pltpu.SemaphoreType.DMA,
            pltpu.SemaphoreType.DMA,
            pltpu.SemaphoreType.DMA,
        ],
    )


def _b1_body(degp_ref, x_ref, w1_ref, hs_ref, dis_ref):
    deg = degp_ref[0] + degp_ref[1] + 1.0
    dis = lax.rsqrt(deg)
    h1 = jnp.dot(x_ref[...], w1_ref[...], preferred_element_type=jnp.float32)
    hs_ref[...] = h1 * dis
    dis_ref[...] = dis


def _b2_body(pp_ref, dis_ref, hs1_ref, w2_ref, b1_ref, hs2_ref):
    dis = dis_ref[...]
    agg = pp_ref[0] + pp_ref[1]
    h = jnp.maximum(dis * agg + dis * hs1_ref[...] + b1_ref[...], 0.0)
    h2 = jnp.dot(h, w2_ref[...], preferred_element_type=jnp.float32)
    hs2_ref[...] = h2 * dis


def _b3_body(qq_ref, dis_ref, hs2_ref, b2_ref, mu_ref, ls_ref):
    dis = dis_ref[...]
    out = dis * (qq_ref[0] + qq_ref[1]) + dis * hs2_ref[...] + b2_ref[...]
    d = mu_ref.shape[1]
    mu_ref[...] = out[:, :d]
    ls_ref[...] = out[:, d:]


def kernel(x, edge_index, W1, b1, W_mu, b_mu, W_ls, b_ls):
    n, d_in = x.shape
    e = edge_index.shape[1]
    d_hid = W1.shape[1]
    d_out = W_mu.shape[1]

    npad = ((n + 255) // 256) * 256
    na = n + _APAD

    rows_pt = _CH * (-(-e // (_EB * _NW * _CH)))
    nrows = _NW * rows_pt
    pad = nrows * _EB - e
    ar = jnp.arange(pad, dtype=jnp.int32)
    src_full = jnp.concatenate([edge_index[0], ar % n])
    dst_full = jnp.concatenate([edge_index[1], n + ar % _APAD])
    src2d = src_full.reshape(nrows, _EB)
    dst2d = dst_full.reshape(nrows, _EB)

    w2 = jnp.concatenate([W_mu, W_ls], axis=1)
    b1r = b1.reshape(1, d_hid)
    b2r = jnp.concatenate([b_mu, b_ls]).reshape(1, 2 * d_out)

    degp = _make_deg(nrows, npad)(dst2d)
    degp3 = degp[:, :n].reshape(_NC, n, 1)

    r = 1000 if n % 1000 == 0 else 8 * (n // 8)
    grid = (n // r,)
    f32 = jnp.float32

    hs1, dis = pl.pallas_call(
        _b1_body,
        grid=grid,
        in_specs=[
            pl.BlockSpec((_NC, r, 1), lambda i: (0, i, 0)),
            pl.BlockSpec((r, d_in), lambda i: (i, 0)),
            pl.BlockSpec((d_in, d_hid), lambda i: (0, 0)),
        ],
        out_specs=[
            pl.BlockSpec((r, d_hid), lambda i: (i, 0)),
            pl.BlockSpec((r, 1), lambda i: (i, 0)),
        ],
        out_shape=[
            jax.ShapeDtypeStruct((n, d_hid), f32),
            jax.ShapeDtypeStruct((n, 1), f32),
        ],
    )(degp3, x, W1)

    pp = _make_agg(na, d_hid, nrows)(hs1, src2d, dst2d)

    hs2 = pl.pallas_call(
        _b2_body,
        grid=grid,
        in_specs=[
            pl.BlockSpec((_NC, r, d_hid), lambda i: (0, i, 0)),
            pl.BlockSpec((r, 1), lambda i: (i, 0)),
            pl.BlockSpec((r, d_hid), lambda i: (i, 0)),
            pl.BlockSpec((d_hid, 2 * d_out), lambda i: (0, 0)),
            pl.BlockSpec((1, d_hid), lambda i: (0, 0)),
        ],
        out_specs=pl.BlockSpec((r, 2 * d_out), lambda i: (i, 0)),
        out_shape=jax.ShapeDtypeStruct((n, 2 * d_out), f32),
    )(pp, dis, hs1, w2, b1r)

    qq = _make_agg(na, 2 * d_out, nrows)(hs2, src2d, dst2d)

    mu, ls = pl.pallas_call(
        _b3_body,
        grid=grid,
        in_specs=[
            pl.BlockSpec((_NC, r, 2 * d_out), lambda i: (0, i, 0)),
            pl.BlockSpec((r, 1), lambda i: (i, 0)),
            pl.BlockSpec((r, 2 * d_out), lambda i: (i, 0)),
            pl.BlockSpec((1, 2 * d_out), lambda i: (0, 0)),
        ],
        out_specs=[
            pl.BlockSpec((r, d_out), lambda i: (i, 0)),
            pl.BlockSpec((r, d_out), lambda i: (i, 0)),
        ],
        out_shape=[
            jax.ShapeDtypeStruct((n, d_out), f32),
            jax.ShapeDtypeStruct((n, d_out), f32),
        ],
    )(qq, dis, hs2, b2r)

    return (mu, ls)


# speedup vs baseline: 1.0980x; 1.0980x over previous
import jax
import jax.numpy as jnp
from jax import lax
from jax.experimental import pallas as pl
from jax.experimental.pallas import tpu as pltpu
from jax.experimental.pallas import tpu_sc as plsc

_NC = 2
_NS = 16
_NW = _NC * _NS
_EB = 128
_CH = 16
_APAD = 240


def _deg_body(dst2d, out, didx, ones_v, zbuf, accum):
    c = lax.axis_index("c")
    s = lax.axis_index("s")
    wid = c * _NS + s
    npad = accum.shape[0]
    rpt = npad // _NS
    rows = dst2d.shape[0] // _NW

    def fill_ones(i, _):
        ones_v[pl.ds(i * 16, 16)] = jnp.ones((16,), jnp.float32)
        return 0

    lax.fori_loop(0, _EB // 16, fill_ones, 0)

    def fill_zero(i, _):
        zbuf[pl.ds(i * 16, 16)] = jnp.zeros((16,), jnp.float32)
        return 0

    lax.fori_loop(0, rpt // 16, fill_zero, 0)

    pltpu.sync_copy(zbuf, accum.at[pl.ds(s * rpt, rpt)])
    pltpu.sync_copy(dst2d.at[pl.ds(wid * rows, rows)], didx)
    plsc.subcore_barrier()

    def body(j, _):
        pltpu.sync_copy(ones_v, accum.at[didx.at[j]], add=True)
        return 0

    lax.fori_loop(0, rows, body, 0)
    plsc.subcore_barrier()
    pltpu.sync_copy(accum.at[pl.ds(s * rpt, rpt)], out.at[c, pl.ds(s * rpt, rpt)])


def _make_deg(nrows, npad):
    mesh = plsc.VectorSubcoreMesh(core_axis_name="c", subcore_axis_name="s")
    return pl.kernel(
        _deg_body,
        out_type=jax.ShapeDtypeStruct((_NC, npad), jnp.float32),
        mesh=mesh,
        scratch_types=[
            pltpu.VMEM((nrows // _NW, _EB), jnp.int32),
            pltpu.VMEM((_EB,), jnp.float32),
            pltpu.VMEM((npad // _NS,), jnp.float32),
            pltpu.VMEM_SHARED((npad,), jnp.float32),
        ],
    )


def _agg_body(hs, src2d, dst2d, out, sc0, sc1, dc0, dc1, b0, b1, accum, s0, s1):
    c = lax.axis_index("c")
    s = lax.axis_index("s")
    wid = c * _NS + s
    na = accum.shape[0]
    rpt = na // _NS
    rows = src2d.shape[0] // _NW
    nch = rows // _CH
    base = wid * rows
    scb = (sc0, sc1)
    dcb = (dc0, dc1)
    bufs = (b0, b1)
    sems = (s0, s1)

    def zfill(i, _):
        for cb in range(8):
            b0[i, pl.ds(cb * 16, 16)] = jnp.zeros((16,), jnp.float32)
        return 0

    lax.fori_loop(0, _EB, zfill, 0)
    for i in range(rpt // _EB):
        pltpu.sync_copy(b0, accum.at[pl.ds(s * rpt + i * _EB, _EB)])
    plsc.subcore_barrier()

    pltpu.sync_copy(src2d.at[pl.ds(base, _CH)], sc0)
    pltpu.sync_copy(dst2d.at[pl.ds(base, _CH)], dc0)
    pltpu.async_copy(hs.at[sc0.at[0]], b0, s0)
    pltpu.async_copy(hs.at[sc0.at[1]], b1, s1)

    for k in range(nch):
        cur_s, cur_d = scb[k % 2], dcb[k % 2]
        nxt_s, nxt_d = scb[(k + 1) % 2], dcb[(k + 1) % 2]
        if k + 1 < nch:
            pltpu.sync_copy(src2d.at[pl.ds(base + (k + 1) * _CH, _CH)], nxt_s)
            pltpu.sync_copy(dst2d.at[pl.ds(base + (k + 1) * _CH, _CH)], nxt_d)
        for jj in range(_CH):
            j = k * _CH + jj
            b = jj % 2
            pltpu.make_async_copy(hs.at[cur_s.at[jj]], bufs[b], sems[b]).wait()
            pltpu.sync_copy(bufs[b], accum.at[cur_d.at[jj]], add=True)
            nj = jj + 2
            if j + 2 < rows:
                if nj < _CH:
                    pltpu.async_copy(hs.at[cur_s.at[nj]], bufs[b], sems[b])
                else:
                    pltpu.async_copy(hs.at[nxt_s.at[nj - _CH]], bufs[b], sems[b])

    plsc.subcore_barrier()
    pltpu.sync_copy(
        accum.at[pl.ds(s * rpt, rpt)], out.at[c, pl.ds(s * rpt, rpt)]
    )


def _make_agg(na, d, nrows):
    mesh = plsc.VectorSubcoreMesh(core_axis_name="c", subcore_axis_name="s")
    return pl.kernel(
        _agg_body,
        out_type=jax.ShapeDtypeStruct((_NC, na, d), jnp.float32),
        mesh=mesh,
        scratch_types=[
            pltpu.VMEM((_CH, _EB), jnp.int32),
            pltpu.VMEM((_CH, _EB), jnp.int32),
            pltpu.VMEM((_CH, _EB), jnp.int32),
            pltpu.VMEM((_CH, _EB), jnp.int32),
            pltpu.VMEM((_EB, d), jnp.float32),
            pltpu.VMEM((_EB, d), jnp.float32),
            pltpu.VMEM_SHARED((na, d), jnp.float32),
            pltpu.SemaphoreType.DMA,
            pltpu.SemaphoreType.DMA,
        ],
    )


def _b1_body(degp_ref, x_ref, w1_ref, hs_ref, dis_ref):
    deg = degp_ref[0] + degp_ref[1] + 1.0
    dis = lax.rsqrt(deg)
    h1 = jnp.dot(x_ref[...], w1_ref[...], preferred_element_type=jnp.float32)
    hs_ref[...] = h1 * dis
    dis_ref[...] = dis


def _b2_body(pp_ref, dis_ref, hs1_ref, w2_ref, b1_ref, hs2_ref):
    dis = dis_ref[...]
    agg = pp_ref[0] + pp_ref[1]
    h = jnp.maximum(dis * agg + dis * hs1_ref[...] + b1_ref[...], 0.0)
    h2 = jnp.dot(h, w2_ref[...], preferred_element_type=jnp.float32)
    hs2_ref[...] = h2 * dis


def _b3_body(qq_ref, dis_ref, hs2_ref, b2_ref, mu_ref, ls_ref):
    dis = dis_ref[...]
    out = dis * (qq_ref[0] + qq_ref[1]) + dis * hs2_ref[...] + b2_ref[...]
    d = mu_ref.shape[1]
    mu_ref[...] = out[:, :d]
    ls_ref[...] = out[:, d:]


def kernel(x, edge_index, W1, b1, W_mu, b_mu, W_ls, b_ls):
    n, d_in = x.shape
    e = edge_index.shape[1]
    d_hid = W1.shape[1]
    d_out = W_mu.shape[1]

    npad = ((n + 255) // 256) * 256
    na = npad

    rows_pt = _CH * (-(-e // (_EB * _NW * _CH)))
    nrows = _NW * rows_pt
    pad = nrows * _EB - e
    apad = min(_APAD, npad - n)
    src_pad = (jnp.arange(pad, dtype=jnp.int32) if pad <= n
               else jnp.arange(pad, dtype=jnp.int32) % n)
    reps = -(-pad // apad)
    dst_pad = (n + jnp.broadcast_to(jnp.arange(apad, dtype=jnp.int32),
                                    (reps, apad)).reshape(-1))[:pad]
    src2d = jnp.concatenate([edge_index[0], src_pad]).reshape(nrows, _EB)
    dst2d = jnp.concatenate([edge_index[1], dst_pad]).reshape(nrows, _EB)

    w2 = jnp.concatenate([W_mu, W_ls], axis=1)
    b1r = b1.reshape(1, d_hid)
    b2r = jnp.concatenate([b_mu, b_ls]).reshape(1, 2 * d_out)

    degp = _make_deg(nrows, npad)(dst2d)
    degp3 = degp[:, :n].reshape(_NC, n, 1)

    r = 2000 if n % 2000 == 0 else 8 * (n // 8)
    grid = (n // r,)
    f32 = jnp.float32

    hs1, dis = pl.pallas_call(
        _b1_body,
        grid=grid,
        in_specs=[
            pl.BlockSpec((_NC, r, 1), lambda i: (0, i, 0)),
            pl.BlockSpec((r, d_in), lambda i: (i, 0)),
            pl.BlockSpec((d_in, d_hid), lambda i: (0, 0)),
        ],
        out_specs=[
            pl.BlockSpec((r, d_hid), lambda i: (i, 0)),
            pl.BlockSpec((r, 1), lambda i: (i, 0)),
        ],
        out_shape=[
            jax.ShapeDtypeStruct((n, d_hid), f32),
            jax.ShapeDtypeStruct((n, 1), f32),
        ],
    )(degp3, x, W1)

    pp = _make_agg(na, d_hid, nrows)(hs1, src2d, dst2d)

    hs2 = pl.pallas_call(
        _b2_body,
        grid=grid,
        in_specs=[
            pl.BlockSpec((_NC, r, d_hid), lambda i: (0, i, 0)),
            pl.BlockSpec((r, 1), lambda i: (i, 0)),
            pl.BlockSpec((r, d_hid), lambda i: (i, 0)),
            pl.BlockSpec((d_hid, 2 * d_out), lambda i: (0, 0)),
            pl.BlockSpec((1, d_hid), lambda i: (0, 0)),
        ],
        out_specs=pl.BlockSpec((r, 2 * d_out), lambda i: (i, 0)),
        out_shape=jax.ShapeDtypeStruct((n, 2 * d_out), f32),
    )(pp, dis, hs1, w2, b1r)

    qq = _make_agg(na, 2 * d_out, nrows)(hs2, src2d, dst2d)

    mu, ls = pl.pallas_call(
        _b3_body,
        grid=grid,
        in_specs=[
            pl.BlockSpec((_NC, r, 2 * d_out), lambda i: (0, i, 0)),
            pl.BlockSpec((r, 1), lambda i: (i, 0)),
            pl.BlockSpec((r, 2 * d_out), lambda i: (i, 0)),
            pl.BlockSpec((1, 2 * d_out), lambda i: (0, 0)),
        ],
        out_specs=[
            pl.BlockSpec((r, d_out), lambda i: (i, 0)),
            pl.BlockSpec((r, d_out), lambda i: (i, 0)),
        ],
        out_shape=[
            jax.ShapeDtypeStruct((n, d_out), f32),
            jax.ShapeDtypeStruct((n, d_out), f32),
        ],
    )(qq, dis, hs2, b2r)

    return (mu, ls)
